# baseline (speedup 1.0000x reference)
import functools

import jax
import jax.numpy as jnp
from jax import lax
from jax.experimental import pallas as pl
from jax.experimental.pallas import tpu as pltpu
from jax.experimental.pallas import tpu_sc as plsc

VOCAB = 100000
EMBED = 64
CTX = 50
HID = 128

VT = 61600
VS = VOCAB - VT

VBLK = 1400
NSTREAM = 4
NBLK = VT // VBLK
NSTEP = NBLK // NSTREAM

NW = 32
RW = VS // NW
CHUNK = 400
NCHUNK = RW // CHUNK
GRP = 16
G_PER_CHUNK = CHUNK // GRP


def _a_body(idx_ref, table_ref, w1_ref, b1_ref, h_ref, sem, emb_ref):
    copies = [
        pltpu.make_async_copy(
            table_ref.at[pl.ds(idx_ref[t], 1), :],
            emb_ref.at[pl.ds(t, 1), :],
            sem,
        )
        for t in range(CTX)
    ]
    for c in copies:
        c.start()
    for c in copies:
        c.wait()
    acc = jnp.zeros((1, HID), jnp.float32)
    for t in range(CTX):
        acc = acc + lax.dot_general(
            emb_ref[t:t + 1, :].astype(jnp.bfloat16),
            w1_ref[:, t * EMBED:(t + 1) * EMBED].astype(jnp.bfloat16),
            (((1,), (1,)), ((), ())), preferred_element_type=jnp.float32)
    h_ref[...] = jnp.maximum(acc + b1_ref[...], 0.0)


_a_call = pl.pallas_call(
    _a_body,
    in_specs=[
        pl.BlockSpec(memory_space=pltpu.SMEM),
        pl.BlockSpec(memory_space=pl.ANY),
        pl.BlockSpec((HID, CTX * EMBED), lambda: (0, 0)),
        pl.BlockSpec((1, HID), lambda: (0, 0)),
    ],
    out_specs=pl.BlockSpec((1, HID), lambda: (0, 0)),
    out_shape=jax.ShapeDtypeStruct((1, HID), jnp.float32),
    scratch_shapes=[
        pltpu.SemaphoreType.DMA,
        pltpu.VMEM((CTX, EMBED), jnp.float32),
    ],
)


def _b_body(h_hbm, w2_hbm, b2_hbm, out_hbm, h_v, chunk_v, b2_v, log_v, sem):
    wid = lax.axis_index("s") * 2 + lax.axis_index("c")
    base = wid * RW
    pltpu.sync_copy(h_hbm, h_v)
    pltpu.sync_copy(b2_hbm.at[pl.ds(base, RW)], b2_v)

    for c in range(NCHUNK):
        pltpu.sync_copy(
            w2_hbm.at[pl.ds((VT + base + c * CHUNK) * HID, CHUNK * HID)],
            chunk_v)

        lane = lax.iota(jnp.int32, GRP)
        h_vecs = [h_v[pl.ds(GRP * i, GRP)] for i in range(HID // GRP)]

        def group(g, carry, c=c):
            row0 = g * GRP
            acc = b2_v[pl.ds(c * CHUNK + row0, GRP)]
            for r in range(GRP):
                addr = (row0 + r) * HID
                prod = chunk_v[pl.ds(addr, GRP)] * h_vecs[0]
                for i in range(1, HID // GRP):
                    prod = prod + chunk_v[pl.ds(addr + GRP * i, GRP)] * h_vecs[i]
                s = jnp.sum(prod)
                acc = jnp.where(lane == r, acc + s, acc)
            log_v[pl.ds(c * CHUNK + row0, GRP)] = acc
            return carry

        lax.fori_loop(0, G_PER_CHUNK, group, 0)

    pltpu.sync_copy(log_v, out_hbm.at[pl.ds(base, RW)])


@functools.cache
def _b_call():
    return pl.kernel(
        _b_body,
        out_type=jax.ShapeDtypeStruct((VS,), jnp.float32),
        mesh=plsc.VectorSubcoreMesh(core_axis_name="c", subcore_axis_name="s"),
        scratch_types=[
            pltpu.VMEM((HID,), jnp.float32),
            pltpu.VMEM((CHUNK * HID,), jnp.float32),
            pltpu.VMEM((RW,), jnp.float32),
            pltpu.VMEM((RW,), jnp.float32),
            pltpu.SemaphoreType.DMA,
        ],
        compiler_params=pltpu.CompilerParams(needs_layout_passes=False),
    )


def _c_body(h_ref, *rest):
    w2_refs = rest[:NSTREAM]
    b2_ref, out_ref = rest[NSTREAM:]
    j = pl.program_id(0)
    hb = h_ref[...].astype(jnp.bfloat16)
    for g in range(NSTREAM):
        logits = lax.dot_general(
            hb, w2_refs[g][...].astype(jnp.bfloat16),
            (((1,), (1,)), ((), ())),
            preferred_element_type=jnp.float32,
        ) + b2_ref[:, 0, g * VBLK:(g + 1) * VBLK]
        out_ref[pl.ds(j * NSTREAM + g, 1)] = logits[None]


def _w2_spec(g):
    return pl.BlockSpec((VBLK, HID), lambda j, g=g: (j * NSTREAM + g, 0))


_c_call = pl.pallas_call(
    _c_body,
    grid=(NSTEP,),
    in_specs=[
        pl.BlockSpec((1, HID), lambda j: (0, 0)),
    ] + [_w2_spec(g) for g in range(NSTREAM)] + [
        pl.BlockSpec((1, 1, NSTREAM * VBLK), lambda j: (j, 0, 0)),
    ],
    out_specs=pl.BlockSpec((NBLK, 1, VBLK), lambda j: (0, 0, 0)),
    out_shape=jax.ShapeDtypeStruct((NBLK, 1, VBLK), jnp.float32),
    compiler_params=pltpu.CompilerParams(
        dimension_semantics=("arbitrary",)),
)


def _d_body(a_ref, b_ref, oa_ref, ob_ref):
    a = a_ref[...]
    b = b_ref[...]
    m = jnp.maximum(jnp.max(a), jnp.max(b))
    s = jnp.sum(jnp.exp(a - m)) + jnp.sum(jnp.exp(b - m))
    lse = m + jnp.log(s)
    oa_ref[...] = a - lse
    ob_ref[...] = b - lse


_d_call = pl.pallas_call(
    _d_body,
    in_specs=[
        pl.BlockSpec((NBLK, 1, VBLK), lambda: (0, 0, 0)),
        pl.BlockSpec((1, VS), lambda: (0, 0)),
    ],
    out_specs=[
        pl.BlockSpec((NBLK, 1, VBLK), lambda: (0, 0, 0)),
        pl.BlockSpec((1, VS), lambda: (0, 0)),
    ],
    out_shape=[
        jax.ShapeDtypeStruct((NBLK, 1, VBLK), jnp.float32),
        jax.ShapeDtypeStruct((1, VS), jnp.float32),
    ],
)


def kernel(inputs, emb_table, W1, b1, W2, b2):
    idx = inputs.astype(jnp.int32)
    h = _a_call(idx, emb_table, W1, b1.reshape(1, HID))
    logits_sc = _b_call()(h.reshape(HID), W2.reshape(VOCAB * HID),
                          lax.slice(b2, (VT,), (VOCAB,)))
    logits_tc = _c_call(
        h,
        *([W2] * NSTREAM),
        lax.slice(b2, (0,), (VT,)).reshape(NSTEP, 1, NSTREAM * VBLK),
    )
    out_a, out_b = _d_call(logits_tc, logits_sc.reshape(1, VS))
    return jnp.concatenate(
        [out_a.reshape(1, VT), out_b], axis=1)


# speedup vs baseline: 1.3369x; 1.3369x over previous
import jax
import jax.numpy as jnp
from jax import lax
from jax.experimental import pallas as pl
from jax.experimental.pallas import tpu as pltpu

VOCAB = 100000
EMBED = 64
CTX = 50
HID = 128

VBLK = 2000
NSTREAM = 5
NBLK = VOCAB // VBLK
NSTEP = NBLK // NSTREAM


def _body(idx_ref, table_ref, w1_ref, b1_ref, *rest):
    w2_refs = rest[:NSTREAM]
    b2_ref, out_ref, sem, emb_ref, h_ref = rest[NSTREAM:]
    j = pl.program_id(0)

    @pl.when(j == 0)
    def _():
        copies = [
            pltpu.make_async_copy(
                table_ref.at[pl.ds(idx_ref[t], 1), :],
                emb_ref.at[pl.ds(t, 1), :],
                sem,
            )
            for t in range(CTX)
        ]
        for c in copies:
            c.start()
        for c in copies:
            c.wait()
        acc = jnp.zeros((1, HID), jnp.float32)
        for t in range(CTX):
            acc = acc + lax.dot_general(
                emb_ref[t:t + 1, :].astype(jnp.bfloat16),
                w1_ref[:, t * EMBED:(t + 1) * EMBED].astype(jnp.bfloat16),
                (((1,), (1,)), ((), ())), preferred_element_type=jnp.float32)
        h_ref[...] = jnp.maximum(acc + b1_ref[...], 0.0).astype(jnp.bfloat16)

    @pl.when(j > 0)
    def _():
        jj = j - 1
        for g in range(NSTREAM):
            logits = lax.dot_general(
                h_ref[...], w2_refs[g][0].astype(jnp.bfloat16),
                (((1,), (1,)), ((), ())),
                preferred_element_type=jnp.float32,
            ) + b2_ref[:, 0, g * VBLK:(g + 1) * VBLK]
            out_ref[pl.ds(jj * NSTREAM + g, 1)] = logits[None]

    @pl.when(j == NSTEP)
    def _():
        x = out_ref[...]
        m = jnp.max(x)
        lse = m + jnp.log(jnp.sum(jnp.exp(x - m)))
        out_ref[...] = x - lse


def _clamp(j):
    return jnp.maximum(j - 1, 0)


def _w2_spec(g):
    return pl.BlockSpec(
        (1, VBLK, HID), lambda j, g=g: (_clamp(j) * NSTREAM + g, 0, 0))


_call = pl.pallas_call(
    _body,
    grid=(NSTEP + 1,),
    in_specs=[
        pl.BlockSpec(memory_space=pltpu.SMEM),
        pl.BlockSpec(memory_space=pl.ANY),
        pl.BlockSpec((HID, CTX * EMBED), lambda j: (0, 0)),
        pl.BlockSpec((1, HID), lambda j: (0, 0)),
    ] + [_w2_spec(g) for g in range(NSTREAM)] + [
        pl.BlockSpec((1, 1, NSTREAM * VBLK), lambda j: (_clamp(j), 0, 0)),
    ],
    out_specs=pl.BlockSpec((NBLK, 1, VBLK), lambda j: (0, 0, 0)),
    out_shape=jax.ShapeDtypeStruct((NBLK, 1, VBLK), jnp.float32),
    scratch_shapes=[
        pltpu.SemaphoreType.DMA,
        pltpu.VMEM((CTX, EMBED), jnp.float32),
        pltpu.VMEM((1, HID), jnp.bfloat16),
    ],
    compiler_params=pltpu.CompilerParams(
        dimension_semantics=("arbitrary",)),
)


def kernel(inputs, emb_table, W1, b1, W2, b2):
    w2r = W2.reshape(NBLK, VBLK, HID)
    out = _call(
        inputs.astype(jnp.int32),
        emb_table,
        W1,
        b1.reshape(1, HID),
        *([w2r] * NSTREAM),
        b2.reshape(NSTEP, 1, NSTREAM * VBLK),
    )
    return out.reshape(1, VOCAB)


# speedup vs baseline: 1.3428x; 1.0044x over previous
import jax
import jax.numpy as jnp
from jax import lax
from jax.experimental import pallas as pl
from jax.experimental.pallas import tpu as pltpu

VOCAB = 100000
EMBED = 64
CTX = 50
HID = 128

VBLK = 4000
NSTREAM = 5
NBLK = VOCAB // VBLK
NSTEP = NBLK // NSTREAM


def _body(idx_ref, table_ref, w1_ref, b1_ref, *rest):
    w2_refs = rest[:NSTREAM]
    b2_ref, out_ref, sem, emb_ref, h_ref = rest[NSTREAM:]
    j = pl.program_id(0)

    @pl.when(j == 0)
    def _():
        copies = [
            pltpu.make_async_copy(
                table_ref.at[pl.ds(idx_ref[t], 1), :],
                emb_ref.at[pl.ds(t, 1), :],
                sem,
            )
            for t in range(CTX)
        ]
        for c in copies:
            c.start()
        for c in copies:
            c.wait()
        acc = jnp.zeros((1, HID), jnp.float32)
        for t in range(CTX):
            acc = acc + lax.dot_general(
                emb_ref[t:t + 1, :].astype(jnp.bfloat16),
                w1_ref[:, t * EMBED:(t + 1) * EMBED].astype(jnp.bfloat16),
                (((1,), (1,)), ((), ())), preferred_element_type=jnp.float32)
        h_ref[...] = jnp.maximum(acc + b1_ref[...], 0.0).astype(jnp.bfloat16)

    for g in range(NSTREAM):
        logits = lax.dot_general(
            h_ref[...], w2_refs[g][0].astype(jnp.bfloat16),
            (((1,), (1,)), ((), ())),
            preferred_element_type=jnp.float32,
        ) + b2_ref[:, 0, g * VBLK:(g + 1) * VBLK]
        out_ref[pl.ds(j * NSTREAM + g, 1)] = logits[None]

    @pl.when(j == NSTEP - 1)
    def _():
        x = out_ref[...]
        m = jnp.max(x)
        lse = m + jnp.log(jnp.sum(jnp.exp(x - m)))
        out_ref[...] = x - lse


def _w2_spec(g):
    return pl.BlockSpec((1, VBLK, HID), lambda j, g=g: (j * NSTREAM + g, 0, 0))


_call = pl.pallas_call(
    _body,
    grid=(NSTEP,),
    in_specs=[
        pl.BlockSpec(memory_space=pltpu.SMEM),
        pl.BlockSpec(memory_space=pl.ANY),
        pl.BlockSpec((HID, CTX * EMBED), lambda j: (0, 0)),
        pl.BlockSpec((1, HID), lambda j: (0, 0)),
    ] + [_w2_spec(g) for g in range(NSTREAM)] + [
        pl.BlockSpec((1, 1, NSTREAM * VBLK), lambda j: (j, 0, 0)),
    ],
    out_specs=pl.BlockSpec((NBLK, 1, VBLK), lambda j: (0, 0, 0)),
    out_shape=jax.ShapeDtypeStruct((NBLK, 1, VBLK), jnp.float32),
    scratch_shapes=[
        pltpu.SemaphoreType.DMA,
        pltpu.VMEM((CTX, EMBED), jnp.float32),
        pltpu.VMEM((1, HID), jnp.bfloat16),
    ],
    compiler_params=pltpu.CompilerParams(
        dimension_semantics=("arbitrary",)),
)


def kernel(inputs, emb_table, W1, b1, W2, b2):
    w2r = W2.reshape(NBLK, VBLK, HID)
    out = _call(
        inputs.astype(jnp.int32),
        emb_table,
        W1,
        b1.reshape(1, HID),
        *([w2r] * NSTREAM),
        b2.reshape(NSTEP, 1, NSTREAM * VBLK),
    )
    return out.reshape(1, VOCAB)
